# baseline (speedup 1.0000x reference)
import functools

import jax
import jax.numpy as jnp
from jax import lax
from jax.experimental import pallas as pl
from jax.experimental.pallas import tpu as pltpu
from jax.experimental.pallas import tpu_sc as plsc

B = 16384
VOCAB = 100000
DIM = 64
LATENT = 128
N_TABLES = 4
CHUNK = 4096


def _make_sc_gather():
    info = plsc.get_sparse_core_info()
    nc, ns = info.num_cores, info.num_subcores
    nw = nc * ns
    cols_per_w = DIM // nw
    mesh = plsc.VectorSubcoreMesh(core_axis_name="c", subcore_axis_name="s")

    @functools.partial(
        pl.kernel,
        mesh=mesh,
        compiler_params=pltpu.CompilerParams(needs_layout_passes=False),
        out_type=[jax.ShapeDtypeStruct((DIM, B // 2), jnp.uint32)] * N_TABLES,
        scratch_types=[
            pltpu.VMEM((VOCAB,), jnp.float32),
            pltpu.VMEM((B,), jnp.int32),
            pltpu.VMEM((CHUNK // 2,), jnp.uint32),
            pltpu.VMEM((CHUNK // 2,), jnp.uint32),
            pltpu.SemaphoreType.DMA,
            pltpu.SemaphoreType.DMA,
        ],
    )
    def gather_all(ig, im, ia, it, tg, tm, ta, tt,
                   og, om, oa, ot, col_v, idx_v, out_a, out_b, sem_a, sem_b):
        wid = lax.axis_index("s") * nc + lax.axis_index("c")
        outs = (out_a, out_b)
        sems = (sem_a, sem_b)
        pending = [None, None]

        def gather_column(col_ref, idx_ref, out_hbm_row, c):
            for ch in range(B // CHUNK):
                buf = (c * (B // CHUNK) + ch) % 2
                if pending[buf] is not None:
                    pending[buf].wait()
                    pending[buf] = None
                out_v = outs[buf]

                @plsc.parallel_loop(0, CHUNK // 32, unroll=8)
                def body(i):
                    va = plsc.load_gather(
                        col_ref, [idx_ref[pl.ds(ch * CHUNK + i * 32, 16)]])
                    vb = plsc.load_gather(
                        col_ref,
                        [idx_ref[pl.ds(ch * CHUNK + i * 32 + 16, 16)]])
                    packed = plsc.pack(
                        va, vb, format=plsc.PackFormat.INTERLEAVED)
                    out_v[pl.ds(i * 16, 16)] = plsc.bitcast(
                        packed, jnp.uint32)

                cp = pltpu.make_async_copy(
                    out_v,
                    out_hbm_row.at[pl.ds(ch * (CHUNK // 2), CHUNK // 2)],
                    sems[buf])
                cp.start()
                pending[buf] = cp

        for idx_hbm, tab_hbm, out_hbm in ((ig, tg, og), (im, tm, om),
                                          (ia, ta, oa), (it, tt, ot)):
            pltpu.sync_copy(idx_hbm, idx_v)
            for c in range(cols_per_w):
                col = wid + c * nw
                pltpu.sync_copy(tab_hbm.at[col], col_v)
                gather_column(col_v, idx_v, out_hbm.at[col], c)
        for buf in range(2):
            if pending[buf] is not None:
                pending[buf].wait()

    return gather_all


_sc_gather_cache = []


def _sc_gather(*args):
    if not _sc_gather_cache:
        _sc_gather_cache.append(_make_sc_gather())
    return _sc_gather_cache[0](*args)


_BLK = 2048


def _mlp_body(eg, em, ea, et, w1, b1, w2, b2, o):
    h = pl.program_id(1)
    dn = (((0,), (0,)), ((), ()))
    f32 = jnp.float32

    def unpack(w):
        lo = w << jnp.uint32(16)
        hi = w & jnp.uint32(0xFFFF0000)
        return lax.bitcast_convert_type(jnp.where(h == 0, lo, hi), f32)

    x = (lax.dot_general(unpack(eg[...]), w1[0 * DIM:1 * DIM, :], dn)
         + lax.dot_general(unpack(em[...]), w1[1 * DIM:2 * DIM, :], dn)
         + lax.dot_general(unpack(ea[...]), w1[2 * DIM:3 * DIM, :], dn)
         + lax.dot_general(unpack(et[...]), w1[3 * DIM:4 * DIM, :], dn))
    hact = jnp.maximum(x + b1[...], 0.0)
    o[...] = hact @ w2[...] + b2[...]


def _mlp(eg, em, ea, et, w1, b1, w2, b2):
    nblocks = (B // 2) // _BLK
    grid = (nblocks, 2)
    e_spec = pl.BlockSpec((DIM, _BLK), lambda i, h: (0, i))
    full = lambda shape: pl.BlockSpec(shape, lambda i, h: (0,) * len(shape))
    return pl.pallas_call(
        _mlp_body,
        grid=grid,
        in_specs=[e_spec, e_spec, e_spec, e_spec,
                  full((N_TABLES * DIM, LATENT)), full((1, LATENT)),
                  full((LATENT, LATENT)), full((1, LATENT))],
        out_specs=pl.BlockSpec((_BLK, LATENT),
                               lambda i, h: (h * nblocks + i, 0)),
        out_shape=jax.ShapeDtypeStruct((B, LATENT), jnp.float32),
    )(eg, em, ea, et, w1, b1, w2, b2)


def _permute_idx(idx):
    idx = idx.astype(jnp.int32)
    return jnp.concatenate(
        [idx[:B // 2].reshape(-1, 16), idx[B // 2:].reshape(-1, 16)],
        axis=1).reshape(B)


def kernel(genre, mood, artist, tempo, table_genre, table_mood,
           table_artist, table_tempo, W1, b1, W2, b2):
    eg, em, ea, et = _sc_gather(
        _permute_idx(genre), _permute_idx(mood),
        _permute_idx(artist), _permute_idx(tempo),
        table_genre.T, table_mood.T, table_artist.T, table_tempo.T)
    eg, em, ea, et = (
        pltpu.with_memory_space_constraint(e, pltpu.HBM)
        for e in (eg, em, ea, et))
    return _mlp(eg, em, ea, et, W1, b1.reshape(1, LATENT), W2,
                b2.reshape(1, LATENT))


# speedup vs baseline: 1.0588x; 1.0588x over previous
import functools

import jax
import jax.numpy as jnp
from jax import lax
from jax.experimental import pallas as pl
from jax.experimental.pallas import tpu as pltpu
from jax.experimental.pallas import tpu_sc as plsc

B = 16384
VOCAB = 100000
DIM = 64
LATENT = 128
N_TABLES = 4
CHUNK = 4096


def _make_sc_gather():
    info = plsc.get_sparse_core_info()
    nc, ns = info.num_cores, info.num_subcores
    nw = nc * ns
    cols_per_w = DIM // nw
    mesh = plsc.VectorSubcoreMesh(core_axis_name="c", subcore_axis_name="s")

    @functools.partial(
        pl.kernel,
        mesh=mesh,
        compiler_params=pltpu.CompilerParams(needs_layout_passes=False),
        out_type=[jax.ShapeDtypeStruct((DIM, B), jnp.float32)] * N_TABLES,
        scratch_types=[
            pltpu.VMEM((VOCAB,), jnp.float32),
            pltpu.VMEM((B,), jnp.int32),
            pltpu.VMEM((CHUNK,), jnp.float32),
            pltpu.VMEM((CHUNK,), jnp.float32),
            pltpu.SemaphoreType.DMA,
            pltpu.SemaphoreType.DMA,
        ],
    )
    def gather_all(ig, im, ia, it, tg, tm, ta, tt,
                   og, om, oa, ot, col_v, idx_v, out_a, out_b, sem_a, sem_b):
        wid = lax.axis_index("s") * nc + lax.axis_index("c")
        outs = (out_a, out_b)
        sems = (sem_a, sem_b)
        pending = [None, None]

        def gather_column(col_ref, idx_ref, out_hbm_row, c):
            for ch in range(B // CHUNK):
                buf = (c * (B // CHUNK) + ch) % 2
                if pending[buf] is not None:
                    pending[buf].wait()
                    pending[buf] = None
                out_v = outs[buf]

                @plsc.parallel_loop(0, CHUNK // 16, unroll=8)
                def body(i):
                    vidx = idx_ref[pl.ds(ch * CHUNK + i * 16, 16)]
                    out_v[pl.ds(i * 16, 16)] = plsc.load_gather(
                        col_ref, [vidx])

                cp = pltpu.make_async_copy(
                    out_v, out_hbm_row.at[pl.ds(ch * CHUNK, CHUNK)],
                    sems[buf])
                cp.start()
                pending[buf] = cp

        for idx_hbm, tab_hbm, out_hbm in ((ig, tg, og), (im, tm, om),
                                          (ia, ta, oa), (it, tt, ot)):
            pltpu.sync_copy(idx_hbm, idx_v)
            for c in range(cols_per_w):
                col = wid + c * nw
                pltpu.sync_copy(tab_hbm.at[col], col_v)
                gather_column(col_v, idx_v, out_hbm.at[col], c)
        for buf in range(2):
            if pending[buf] is not None:
                pending[buf].wait()

    return gather_all


_sc_gather_cache = []


def _sc_gather(*args):
    if not _sc_gather_cache:
        _sc_gather_cache.append(_make_sc_gather())
    return _sc_gather_cache[0](*args)


_BLK = 2048


def _mlp_body(eg, em, ea, et, w1, b1, w2, b2, o):
    dn = (((0,), (0,)), ((), ()))
    x = (lax.dot_general(eg[...], w1[0 * DIM:1 * DIM, :], dn)
         + lax.dot_general(em[...], w1[1 * DIM:2 * DIM, :], dn)
         + lax.dot_general(ea[...], w1[2 * DIM:3 * DIM, :], dn)
         + lax.dot_general(et[...], w1[3 * DIM:4 * DIM, :], dn))
    h = jnp.maximum(x + b1[...], 0.0)
    o[...] = h @ w2[...] + b2[...]


def _mlp(eg, em, ea, et, w1, b1, w2, b2):
    grid = (B // _BLK,)
    e_spec = pl.BlockSpec((DIM, _BLK), lambda i: (0, i))
    full = lambda shape: pl.BlockSpec(shape, lambda i: (0,) * len(shape))
    return pl.pallas_call(
        _mlp_body,
        grid=grid,
        in_specs=[e_spec, e_spec, e_spec, e_spec,
                  full((N_TABLES * DIM, LATENT)), full((1, LATENT)),
                  full((LATENT, LATENT)), full((1, LATENT))],
        out_specs=pl.BlockSpec((_BLK, LATENT), lambda i: (i, 0)),
        out_shape=jax.ShapeDtypeStruct((B, LATENT), jnp.float32),
    )(eg, em, ea, et, w1, b1, w2, b2)


def kernel(genre, mood, artist, tempo, table_genre, table_mood,
           table_artist, table_tempo, W1, b1, W2, b2):
    eg, em, ea, et = _sc_gather(
        genre.astype(jnp.int32), mood.astype(jnp.int32),
        artist.astype(jnp.int32), tempo.astype(jnp.int32),
        table_genre.T, table_mood.T, table_artist.T, table_tempo.T)
    eg, em, ea, et = (
        pltpu.with_memory_space_constraint(e, pltpu.HBM)
        for e in (eg, em, ea, et))
    return _mlp(eg, em, ea, et, W1, b1.reshape(1, LATENT), W2,
                b2.reshape(1, LATENT))
